# Initial kernel scaffold; baseline (speedup 1.0000x reference)
import jax
import jax.numpy as jnp
from jax.experimental import pallas as pl


def kernel(x, W1, b1, W2, b2, W3, b3, codebook, D1, c1, D2, c2, D3, c3):
    raise NotImplementedError("write your pallas kernel here")


# speedup vs baseline: 1.1683x; 1.1683x over previous
import functools

import jax
import jax.numpy as jnp
from jax import lax
from jax.experimental import pallas as pl
from jax.experimental.pallas import tpu as pltpu
from jax.experimental.pallas import tpu_sc as plsc

B = 8192
INPUT_DIM = 768
LATENT_DIM = 64
NUM_EMB = 8192

BT = 512
NB = B // BT
CHUNK = 2048
NCHUNK = NUM_EMB // CHUNK


def _dot(a, b, dims):
    return lax.dot_general(a, b, (dims, ((), ())),
                           preferred_element_type=jnp.float32)


def _enc_vq_body(x_ref, W1_ref, b1_ref, W2_ref, b2_ref, W3_ref, b3_ref,
                 cb_ref, z_ref, idx_ref):
    x = x_ref[...]
    h = jnp.maximum(_dot(x, W1_ref[...], ((1,), (0,))) + b1_ref[...], 0.0)
    h = jnp.maximum(_dot(h, W2_ref[...], ((1,), (0,))) + b2_ref[...], 0.0)
    z = _dot(h, W3_ref[...], ((1,), (0,))) + b3_ref[...]
    z_ref[...] = z

    zz = jnp.sum(z * z, axis=1, keepdims=True)
    best = jnp.full((BT,), jnp.inf, dtype=jnp.float32)
    besti = jnp.zeros((BT,), dtype=jnp.int32)
    for j in range(NCHUNK):
        cb = cb_ref[j * CHUNK:(j + 1) * CHUNK, :]
        n2 = jnp.sum(cb * cb, axis=1)
        d = zz - 2.0 * _dot(z, cb, ((1,), (1,))) + n2[None, :]
        lmin = jnp.min(d, axis=1)
        col = lax.broadcasted_iota(jnp.int32, (BT, CHUNK), 1)
        lidx = jnp.min(jnp.where(d == lmin[:, None], col, NUM_EMB), axis=1)
        upd = lmin < best
        best = jnp.where(upd, lmin, best)
        besti = jnp.where(upd, lidx + j * CHUNK, besti)
    idx_ref[0, 0, :] = besti


def _dec_body(z_ref, zq_ref, D1_ref, c1_ref, D2_ref, c2_ref, D3_ref, c3_ref,
              xr_ref, loss_ref):
    z = z_ref[...]
    zq = zq_ref[:, :LATENT_DIM]
    zst = z + (zq - z)
    h = jnp.maximum(_dot(zst, D1_ref[...], ((1,), (0,))) + c1_ref[...], 0.0)
    h = jnp.maximum(_dot(h, D2_ref[...], ((1,), (0,))) + c2_ref[...], 0.0)
    xr_ref[...] = jnp.tanh(_dot(h, D3_ref[...], ((1,), (0,))) + c3_ref[...])

    part = jnp.sum((zq - z) ** 2).reshape(1, 1)

    @pl.when(pl.program_id(0) == 0)
    def _init():
        loss_ref[...] = part

    @pl.when(pl.program_id(0) != 0)
    def _acc():
        loss_ref[...] += part


def _const_spec(shape):
    return pl.BlockSpec(shape, lambda i: (0,) * len(shape))


GD = 128


def _sc_gather(codebook_padded, idx):
    info = plsc.get_sparse_core_info()
    nw = info.num_cores * info.num_subcores
    bpw = B // nw
    mesh = plsc.VectorSubcoreMesh(core_axis_name="c", subcore_axis_name="s")

    @functools.partial(
        pl.kernel, mesh=mesh,
        out_type=jax.ShapeDtypeStruct((B, GD), jnp.float32),
        scratch_types=[
            pltpu.VMEM((bpw,), jnp.int32),
            pltpu.VMEM((bpw, GD), jnp.float32),
            pltpu.SemaphoreType.DMA,
        ],
    )
    def gather_k(table_hbm, idx_hbm, out_hbm, idx_v, rows_v, sem):
        wid = lax.axis_index("s") * info.num_cores + lax.axis_index("c")
        base = wid * bpw
        pltpu.sync_copy(idx_hbm.at[pl.ds(base, bpw)], idx_v)
        pltpu.async_copy(table_hbm.at[idx_v], rows_v, sem).wait()
        pltpu.sync_copy(rows_v, out_hbm.at[pl.ds(base, bpw)])

    return gather_k(codebook_padded, idx)


def kernel(x, W1, b1, W2, b2, W3, b3, codebook, D1, c1, D2, c2, D3, c3):
    z, idx3 = pl.pallas_call(
        _enc_vq_body,
        grid=(NB,),
        in_specs=[
            pl.BlockSpec((BT, INPUT_DIM), lambda i: (i, 0)),
            _const_spec((INPUT_DIM, 512)),
            _const_spec((1, 512)),
            _const_spec((512, 256)),
            _const_spec((1, 256)),
            _const_spec((256, LATENT_DIM)),
            _const_spec((1, LATENT_DIM)),
            _const_spec((NUM_EMB, LATENT_DIM)),
        ],
        out_specs=[
            pl.BlockSpec((BT, LATENT_DIM), lambda i: (i, 0)),
            pl.BlockSpec((1, 1, BT), lambda i: (i, 0, 0)),
        ],
        out_shape=[
            jax.ShapeDtypeStruct((B, LATENT_DIM), jnp.float32),
            jax.ShapeDtypeStruct((NB, 1, BT), jnp.int32),
        ],
    )(x, W1, b1.reshape(1, -1), W2, b2.reshape(1, -1), W3,
      b3.reshape(1, -1), codebook)

    idx = idx3.reshape(B)
    cb_pad = jnp.pad(codebook, ((0, 0), (0, GD - LATENT_DIM)))
    zq = _sc_gather(cb_pad, idx)

    xr, loss = pl.pallas_call(
        _dec_body,
        grid=(NB,),
        in_specs=[
            pl.BlockSpec((BT, LATENT_DIM), lambda i: (i, 0)),
            pl.BlockSpec((BT, GD), lambda i: (i, 0)),
            _const_spec((LATENT_DIM, 256)),
            _const_spec((1, 256)),
            _const_spec((256, 512)),
            _const_spec((1, 512)),
            _const_spec((512, INPUT_DIM)),
            _const_spec((1, INPUT_DIM)),
        ],
        out_specs=[
            pl.BlockSpec((BT, INPUT_DIM), lambda i: (i, 0)),
            _const_spec((1, 1)),
        ],
        out_shape=[
            jax.ShapeDtypeStruct((B, INPUT_DIM), jnp.float32),
            jax.ShapeDtypeStruct((1, 1), jnp.float32),
        ],
    )(z, zq, D1, c1.reshape(1, -1), D2, c2.reshape(1, -1), D3,
      c3.reshape(1, -1))

    commitment_loss = 0.25 * (loss[0, 0] / (B * LATENT_DIM))
    return (xr, z, idx, commitment_loss)
